# baseline (speedup 1.0000x reference)
import functools

import jax
import jax.numpy as jnp
from jax import lax
from jax.experimental import pallas as pl
from jax.experimental.pallas import tpu as pltpu
from jax.experimental.pallas import tpu_sc as plsc

N = 100000
E = 3200000
H = 20
G = 512
NCLS = 2

FH = 16
NC = 2
NS = 16
LB = 128
K = 5
TOTAL_BLOCKS = 25120
BPT = TOTAL_BLOCKS // NS
OUTER = BPT // K
E_PAD = TOTAL_BLOCKS * LB
N_PAD = 100352
RPT = N_PAD // NS

PH = N * FH // LB
PB = 448
NB = -(-PH // PB)
PAGG = NC * N_PAD * FH // LB
AOFF = N_PAD * FH // LB
BN = 2048
NBP = -(-N // BN)


def _make_sc_agg():
  mesh = plsc.VectorSubcoreMesh(core_axis_name="c", subcore_axis_name="s")

  @functools.partial(
      pl.kernel,
      mesh=mesh,
      compiler_params=pltpu.CompilerParams(use_tc_tiling_on_sc=False),
      out_type=jax.ShapeDtypeStruct((NC * N_PAD, FH), jnp.float32),
      scratch_types=[
          pltpu.VMEM((3, K, LB), jnp.int32),
          pltpu.VMEM((3, K, LB), jnp.int32),
          pltpu.VMEM((2, K, LB, FH), jnp.float32),
          pltpu.VMEM_SHARED((N_PAD, FH), jnp.float32),
          pltpu.SemaphoreType.DMA,
          pltpu.SemaphoreType.DMA,
          pltpu.SemaphoreType.DMA,
      ],
  )
  def agg(ha_hbm, hb_hbm, src_hbm, dst_hbm, zr_hbm, out_hbm,
          src_v, dst_v, rows_v, acc_sh, gsem, isem, ssem):
    c = lax.axis_index("c")
    s = lax.axis_index("s")
    row0 = s * RPT

    pltpu.sync_copy(zr_hbm, acc_sh.at[pl.ds(row0, RPT)])
    plsc.subcore_barrier()

    blk0 = s * BPT

    def run_edges(h_hbm):
      def idx_copy(g, q, sem):
        base = blk0 + g * K
        return (
            pltpu.make_async_copy(src_hbm.at[pl.ds(base, K)],
                                  src_v.at[q], sem),
            pltpu.make_async_copy(dst_hbm.at[pl.ds(base, K)],
                                  dst_v.at[q], sem),
        )

      def gather(q, p):
        return [pltpu.make_async_copy(h_hbm.at[src_v.at[q, j]],
                                      rows_v.at[p, j], gsem)
                for j in range(K)]

      for cp in idx_copy(0, 0, isem):
        cp.start()
        cp.wait()
      cps0 = gather(0, 0)
      for cp in cps0:
        cp.start()
      for cp in idx_copy(1, 1, isem):
        cp.start()
      for cp in cps0:
        cp.wait()

      def ebody(g, carry):
        p = lax.rem(g, 2)
        pn = 1 - p
        q0 = lax.rem(g, 3)
        q1 = lax.rem(g + 1, 3)
        q2 = lax.rem(g + 2, 3)

        @pl.when(g + 1 < OUTER)
        def _():
          for cp in idx_copy(g + 1, q1, isem):
            cp.wait()
          for cp in gather(q1, pn):
            cp.start()

        @pl.when(g + 2 < OUTER)
        def _():
          for cp in idx_copy(g + 2, q2, isem):
            cp.start()

        scats = [pltpu.make_async_copy(rows_v.at[p, j],
                                       acc_sh.at[dst_v.at[q0, j]], ssem)
                 for j in range(K)]
        for cp in scats:
          cp.start(add=True)

        @pl.when(g + 1 < OUTER)
        def _():
          for cp in gather(q1, pn):
            cp.wait()

        for cp in scats:
          cp.wait()
        return carry
      lax.fori_loop(0, OUTER, ebody, 0)

    @pl.when(c == 0)
    def _():
      run_edges(ha_hbm)

    @pl.when(c == 1)
    def _():
      run_edges(hb_hbm)

    plsc.subcore_barrier()

    pltpu.sync_copy(acc_sh.at[pl.ds(row0, RPT)],
                    out_hbm.at[pl.ds(c * N_PAD + row0, RPT)])

  return agg


def _dense1(ha_p, hb_p, agg_p, kw1, b1r, kw2, b2r):
  def body(ha_ref, hb_ref, a0_ref, a1_ref, kw1_ref, b1_ref, kw2_ref, b2_ref,
           za_ref, zb_ref, st_ref):
    i = pl.program_id(0)
    pa = ha_ref[...] + a0_ref[...]
    pb = hb_ref[...] + a1_ref[...]

    def mm(xa, xb, kw_ref, b_ref):
      ya = (jnp.dot(xa, kw_ref[0], preferred_element_type=jnp.float32)
            + jnp.dot(xb, kw_ref[1], preferred_element_type=jnp.float32)
            + b_ref[0:1])
      yb = (jnp.dot(xa, kw_ref[2], preferred_element_type=jnp.float32)
            + jnp.dot(xb, kw_ref[3], preferred_element_type=jnp.float32)
            + b_ref[1:2])
      return ya, yb

    z1a, z1b = mm(pa, pb, kw1_ref, b1_ref)
    z1a = jnp.maximum(z1a, 0.0)
    z1b = jnp.maximum(z1b, 0.0)
    za, zb = mm(z1a, z1b, kw2_ref, b2_ref)
    za_ref[...] = za
    zb_ref[...] = zb
    valid = (lax.broadcasted_iota(jnp.int32, (PB, 1), 0) + i * PB) < PH
    zam = jnp.where(valid, za, 0.0)
    zbm = jnp.where(valid, zb, 0.0)
    st = jnp.concatenate(
        [jnp.sum(zam, axis=0, keepdims=True),
         jnp.sum(zam * zam, axis=0, keepdims=True),
         jnp.sum(zbm, axis=0, keepdims=True),
         jnp.sum(zbm * zbm, axis=0, keepdims=True),
         jnp.zeros((4, LB), jnp.float32)], axis=0)

    @pl.when(i == 0)
    def _():
      st_ref[...] = st

    @pl.when(i > 0)
    def _():
      st_ref[...] = st_ref[...] + st

  return pl.pallas_call(
      body,
      grid=(NB,),
      in_specs=[
          pl.BlockSpec((PB, LB), lambda i: (i, 0)),
          pl.BlockSpec((PB, LB), lambda i: (i, 0)),
          pl.BlockSpec((PB, LB), lambda i: (i, 0)),
          pl.BlockSpec((PB, LB), lambda i: (AOFF // PB + i, 0)),
          pl.BlockSpec((4, LB, LB), lambda i: (0, 0, 0)),
          pl.BlockSpec((2, LB), lambda i: (0, 0)),
          pl.BlockSpec((4, LB, LB), lambda i: (0, 0, 0)),
          pl.BlockSpec((2, LB), lambda i: (0, 0)),
      ],
      out_specs=[
          pl.BlockSpec((PB, LB), lambda i: (i, 0)),
          pl.BlockSpec((PB, LB), lambda i: (i, 0)),
          pl.BlockSpec((8, LB), lambda i: (0, 0)),
      ],
      out_shape=[
          jax.ShapeDtypeStruct((PH, LB), jnp.float32),
          jax.ShapeDtypeStruct((PH, LB), jnp.float32),
          jax.ShapeDtypeStruct((8, LB), jnp.float32),
      ],
  )(ha_p, hb_p, agg_p, agg_p, kw1, b1r, kw2, b2r)


def _fold(row):
  t = row[:, 0:16]
  for k in range(1, 8):
    t = t + row[:, 16 * k:16 * k + 16]
  return jnp.concatenate([t] * 8, axis=1)


def _bn_relu(za_p, zb_p, st, gr, br):
  def body(za_ref, zb_ref, st_ref, g_ref, b_ref, ha_ref, hb_ref):
    n_inv = 1.0 / N
    mua = _fold(st_ref[0:1]) * n_inv
    ex2a = _fold(st_ref[1:2]) * n_inv
    mub = _fold(st_ref[2:3]) * n_inv
    ex2b = _fold(st_ref[3:4]) * n_inv
    inva = lax.rsqrt(ex2a - mua * mua + 1e-5)
    invb = lax.rsqrt(ex2b - mub * mub + 1e-5)
    ha_ref[...] = jnp.maximum(
        g_ref[0:1] * (za_ref[...] - mua) * inva + b_ref[0:1], 0.0)
    hb_ref[...] = jnp.maximum(
        g_ref[1:2] * (zb_ref[...] - mub) * invb + b_ref[1:2], 0.0)

  return pl.pallas_call(
      body,
      grid=(NB,),
      in_specs=[
          pl.BlockSpec((PB, LB), lambda i: (i, 0)),
          pl.BlockSpec((PB, LB), lambda i: (i, 0)),
          pl.BlockSpec((8, LB), lambda i: (0, 0)),
          pl.BlockSpec((2, LB), lambda i: (0, 0)),
          pl.BlockSpec((2, LB), lambda i: (0, 0)),
      ],
      out_specs=[
          pl.BlockSpec((PB, LB), lambda i: (i, 0)),
          pl.BlockSpec((PB, LB), lambda i: (i, 0)),
      ],
      out_shape=[
          jax.ShapeDtypeStruct((PH, LB), jnp.float32),
          jax.ShapeDtypeStruct((PH, LB), jnp.float32),
      ],
  )(za_p, zb_p, st, gr, br)


def _pool(h2, batch3d):
  def body(h_ref, bat_ref, gs_ref):
    i = pl.program_id(0)
    valid = (lax.broadcasted_iota(jnp.int32, (BN, 1), 0) + i * BN) < N
    hnm = jnp.where(valid, h_ref[...], 0.0)
    bid = bat_ref[0, 0, :]
    oh = (lax.broadcasted_iota(jnp.int32, (G, BN), 0)
          == bid[None, :]).astype(jnp.float32)
    p = jnp.dot(oh, hnm, preferred_element_type=jnp.float32)

    @pl.when(i == 0)
    def _():
      gs_ref[...] = p

    @pl.when(i > 0)
    def _():
      gs_ref[...] = gs_ref[...] + p

  return pl.pallas_call(
      body,
      grid=(NBP,),
      in_specs=[
          pl.BlockSpec((BN, H), lambda i: (i, 0)),
          pl.BlockSpec((1, 1, BN), lambda i: (i, 0, 0)),
      ],
      out_specs=pl.BlockSpec((G, H), lambda i: (0, 0)),
      out_shape=jax.ShapeDtypeStruct((G, H), jnp.float32),
  )(h2, batch3d)


def _fc(gsum, fcw, fcb):
  def body(g_ref, w_ref, b_ref, o_ref):
    o_ref[...] = (jnp.dot(g_ref[...], w_ref[...],
                          preferred_element_type=jnp.float32) + b_ref[...])

  return pl.pallas_call(
      body,
      out_shape=jax.ShapeDtypeStruct((G, NCLS), jnp.float32),
  )(gsum, fcw, fcb)


def kernel(x, edge_index, batch, params):
  src = edge_index[0]
  dst = edge_index[1]
  npad = E_PAD - E
  pad_src = jnp.arange(npad, dtype=jnp.int32) % LB
  pad_dst = N + jnp.arange(npad, dtype=jnp.int32) % (N_PAD - N)
  src2d = jnp.concatenate([src, pad_src]).reshape(TOTAL_BLOCKS, LB)
  dst2d = jnp.concatenate([dst, pad_dst]).reshape(TOTAL_BLOCKS, LB)

  xa_p = jnp.concatenate(
      [x, jnp.zeros((N, FH - 7), jnp.float32)], axis=1).reshape(PH, LB)
  xb_p = jnp.zeros((PH, LB), jnp.float32)
  zfill = jnp.zeros((RPT, FH), jnp.float32)
  batch3d = jnp.concatenate(
      [batch, jnp.zeros((NBP * BN - N,), batch.dtype)]).reshape(
          NBP, 1, BN).astype(jnp.int32)

  eye8 = jnp.eye(8, dtype=jnp.float32)

  def halves(w):
    waa = w[:FH, :FH]
    wba = jnp.zeros((FH, FH), jnp.float32).at[:H - FH, :].set(w[FH:, :FH])
    wab = jnp.zeros((FH, FH), jnp.float32).at[:, :H - FH].set(w[:FH, FH:])
    wbb = jnp.zeros((FH, FH), jnp.float32).at[:H - FH, :H - FH].set(
        w[FH:, FH:])
    return jnp.stack([jnp.kron(eye8, m) for m in (waa, wba, wab, wbb)])

  def btile(b):
    ba = jnp.tile(b[:FH], 8)
    bb = jnp.tile(jnp.concatenate([b[FH:], jnp.zeros((2 * FH - H,),
                                                     jnp.float32)]), 8)
    return jnp.stack([ba, bb])

  w1p0 = jnp.concatenate(
      [params['W1_0'], jnp.zeros((H - 7, H), jnp.float32)], axis=0)
  kw1s = jnp.stack([halves(w1p0), halves(params['W1_1']),
                    halves(params['W1_2'])])
  kw2s = jnp.stack([halves(params[f'W2_{i}']) for i in range(3)])
  b1s = jnp.stack([btile(params[f'b1_{i}']) for i in range(3)])
  b2s = jnp.stack([btile(params[f'b2_{i}']) for i in range(3)])
  gms = jnp.stack([btile(params[f'bn_gamma_{i}']) for i in range(3)])
  bts = jnp.stack([btile(params[f'bn_beta_{i}']) for i in range(3)])

  agg = _make_sc_agg()

  def layer(h, ws):
    ha_p, hb_p = h
    kw1, b1r, kw2, b2r, gr, br = ws
    a = agg(ha_p.reshape(N, FH), hb_p.reshape(N, FH), src2d, dst2d, zfill)
    za_p, zb_p, st = _dense1(ha_p, hb_p, a.reshape(PAGG, LB),
                             kw1, b1r, kw2, b2r)
    return _bn_relu(za_p, zb_p, st, gr, br), None

  (ha2, hb2), _ = lax.scan(layer, (xa_p, xb_p), (kw1s, b1s, kw2s, b2s,
                                                 gms, bts))

  h2 = jnp.concatenate([ha2.reshape(N, FH),
                        hb2.reshape(N, FH)[:, :H - FH]], axis=1)
  gsum = _pool(h2, batch3d)
  out = _fc(gsum, params['fc_W'], params['fc_b'].reshape(1, NCLS))
  return (out, h2, gsum)


# speedup vs baseline: 1.0075x; 1.0075x over previous
import functools

import jax
import jax.numpy as jnp
from jax import lax
from jax.experimental import pallas as pl
from jax.experimental.pallas import tpu as pltpu
from jax.experimental.pallas import tpu_sc as plsc

N = 100000
E = 3200000
H = 20
G = 512
NCLS = 2

FH = 16
NC = 2
NS = 16
LB = 128
K = 5
TOTAL_BLOCKS = 25120
BPT = TOTAL_BLOCKS // NS
OUTER = BPT // K
E_PAD = TOTAL_BLOCKS * LB
N_PAD = 100352
RPT = N_PAD // NS

PH = N * FH // LB
PB = 448
NB = -(-PH // PB)
PAGG = NC * N_PAD * FH // LB
AOFF = N_PAD * FH // LB
BN = 2048
NBP = -(-N // BN)


def _make_sc_agg():
  mesh = plsc.VectorSubcoreMesh(core_axis_name="c", subcore_axis_name="s")

  @functools.partial(
      pl.kernel,
      mesh=mesh,
      compiler_params=pltpu.CompilerParams(use_tc_tiling_on_sc=False),
      out_type=jax.ShapeDtypeStruct((NC * N_PAD, FH), jnp.float32),
      scratch_types=[
          pltpu.VMEM((3, K, LB), jnp.int32),
          pltpu.VMEM((3, K, LB), jnp.int32),
          pltpu.VMEM((2, K, LB, FH), jnp.float32),
          pltpu.VMEM_SHARED((N_PAD, FH), jnp.float32),
          pltpu.SemaphoreType.DMA,
          pltpu.SemaphoreType.DMA,
      ],
  )
  def agg(ha_hbm, hb_hbm, src_hbm, dst_hbm, zr_hbm, out_hbm,
          src_v, dst_v, rows_v, acc_sh, gsem, isem):
    c = lax.axis_index("c")
    s = lax.axis_index("s")
    row0 = s * RPT

    pltpu.sync_copy(zr_hbm, acc_sh.at[pl.ds(row0, RPT)])
    plsc.subcore_barrier()

    blk0 = s * BPT

    def run_edges(h_hbm):
      def idx_copy(g, q, sem):
        base = blk0 + g * K
        return (
            pltpu.make_async_copy(src_hbm.at[pl.ds(base, K)],
                                  src_v.at[q], sem),
            pltpu.make_async_copy(dst_hbm.at[pl.ds(base, K)],
                                  dst_v.at[q], sem),
        )

      def gather(q, p):
        return [pltpu.make_async_copy(h_hbm.at[src_v.at[q, j]],
                                      rows_v.at[p, j], gsem)
                for j in range(K)]

      for cp in idx_copy(0, 0, isem):
        cp.start()
        cp.wait()
      cps0 = gather(0, 0)
      for cp in cps0:
        cp.start()
      for cp in idx_copy(1, 1, isem):
        cp.start()
      for cp in cps0:
        cp.wait()

      def ebody(g, carry):
        p = lax.rem(g, 2)
        pn = 1 - p
        q0 = lax.rem(g, 3)
        q1 = lax.rem(g + 1, 3)
        q2 = lax.rem(g + 2, 3)

        @pl.when(g + 1 < OUTER)
        def _():
          for cp in idx_copy(g + 1, q1, isem):
            cp.wait()
          for cp in gather(q1, pn):
            cp.start()

        @pl.when(g + 2 < OUTER)
        def _():
          for cp in idx_copy(g + 2, q2, isem):
            cp.start()

        for j in range(K):
          pltpu.sync_copy(rows_v.at[p, j], acc_sh.at[dst_v.at[q0, j]],
                          add=True)

        @pl.when(g + 1 < OUTER)
        def _():
          for cp in gather(q1, pn):
            cp.wait()
        return carry
      lax.fori_loop(0, OUTER, ebody, 0)

    @pl.when(c == 0)
    def _():
      run_edges(ha_hbm)

    @pl.when(c == 1)
    def _():
      run_edges(hb_hbm)

    plsc.subcore_barrier()

    pltpu.sync_copy(acc_sh.at[pl.ds(row0, RPT)],
                    out_hbm.at[pl.ds(c * N_PAD + row0, RPT)])

  return agg


def _dense1(ha_p, hb_p, agg_p, kw1, b1r, kw2, b2r):
  def body(ha_ref, hb_ref, a0_ref, a1_ref, kw1_ref, b1_ref, kw2_ref, b2_ref,
           za_ref, zb_ref, st_ref):
    i = pl.program_id(0)
    pa = ha_ref[...] + a0_ref[...]
    pb = hb_ref[...] + a1_ref[...]

    def mm(xa, xb, kw_ref, b_ref):
      ya = (jnp.dot(xa, kw_ref[0], preferred_element_type=jnp.float32)
            + jnp.dot(xb, kw_ref[1], preferred_element_type=jnp.float32)
            + b_ref[0:1])
      yb = (jnp.dot(xa, kw_ref[2], preferred_element_type=jnp.float32)
            + jnp.dot(xb, kw_ref[3], preferred_element_type=jnp.float32)
            + b_ref[1:2])
      return ya, yb

    z1a, z1b = mm(pa, pb, kw1_ref, b1_ref)
    z1a = jnp.maximum(z1a, 0.0)
    z1b = jnp.maximum(z1b, 0.0)
    za, zb = mm(z1a, z1b, kw2_ref, b2_ref)
    za_ref[...] = za
    zb_ref[...] = zb
    valid = (lax.broadcasted_iota(jnp.int32, (PB, 1), 0) + i * PB) < PH
    zam = jnp.where(valid, za, 0.0)
    zbm = jnp.where(valid, zb, 0.0)
    st = jnp.concatenate(
        [jnp.sum(zam, axis=0, keepdims=True),
         jnp.sum(zam * zam, axis=0, keepdims=True),
         jnp.sum(zbm, axis=0, keepdims=True),
         jnp.sum(zbm * zbm, axis=0, keepdims=True),
         jnp.zeros((4, LB), jnp.float32)], axis=0)

    @pl.when(i == 0)
    def _():
      st_ref[...] = st

    @pl.when(i > 0)
    def _():
      st_ref[...] = st_ref[...] + st

  return pl.pallas_call(
      body,
      grid=(NB,),
      in_specs=[
          pl.BlockSpec((PB, LB), lambda i: (i, 0)),
          pl.BlockSpec((PB, LB), lambda i: (i, 0)),
          pl.BlockSpec((PB, LB), lambda i: (i, 0)),
          pl.BlockSpec((PB, LB), lambda i: (AOFF // PB + i, 0)),
          pl.BlockSpec((4, LB, LB), lambda i: (0, 0, 0)),
          pl.BlockSpec((2, LB), lambda i: (0, 0)),
          pl.BlockSpec((4, LB, LB), lambda i: (0, 0, 0)),
          pl.BlockSpec((2, LB), lambda i: (0, 0)),
      ],
      out_specs=[
          pl.BlockSpec((PB, LB), lambda i: (i, 0)),
          pl.BlockSpec((PB, LB), lambda i: (i, 0)),
          pl.BlockSpec((8, LB), lambda i: (0, 0)),
      ],
      out_shape=[
          jax.ShapeDtypeStruct((PH, LB), jnp.float32),
          jax.ShapeDtypeStruct((PH, LB), jnp.float32),
          jax.ShapeDtypeStruct((8, LB), jnp.float32),
      ],
  )(ha_p, hb_p, agg_p, agg_p, kw1, b1r, kw2, b2r)


def _fold(row):
  t = row[:, 0:16]
  for k in range(1, 8):
    t = t + row[:, 16 * k:16 * k + 16]
  return jnp.concatenate([t] * 8, axis=1)


def _bn_relu(za_p, zb_p, st, gr, br):
  def body(za_ref, zb_ref, st_ref, g_ref, b_ref, ha_ref, hb_ref):
    n_inv = 1.0 / N
    mua = _fold(st_ref[0:1]) * n_inv
    ex2a = _fold(st_ref[1:2]) * n_inv
    mub = _fold(st_ref[2:3]) * n_inv
    ex2b = _fold(st_ref[3:4]) * n_inv
    inva = lax.rsqrt(ex2a - mua * mua + 1e-5)
    invb = lax.rsqrt(ex2b - mub * mub + 1e-5)
    ha_ref[...] = jnp.maximum(
        g_ref[0:1] * (za_ref[...] - mua) * inva + b_ref[0:1], 0.0)
    hb_ref[...] = jnp.maximum(
        g_ref[1:2] * (zb_ref[...] - mub) * invb + b_ref[1:2], 0.0)

  return pl.pallas_call(
      body,
      grid=(NB,),
      in_specs=[
          pl.BlockSpec((PB, LB), lambda i: (i, 0)),
          pl.BlockSpec((PB, LB), lambda i: (i, 0)),
          pl.BlockSpec((8, LB), lambda i: (0, 0)),
          pl.BlockSpec((2, LB), lambda i: (0, 0)),
          pl.BlockSpec((2, LB), lambda i: (0, 0)),
      ],
      out_specs=[
          pl.BlockSpec((PB, LB), lambda i: (i, 0)),
          pl.BlockSpec((PB, LB), lambda i: (i, 0)),
      ],
      out_shape=[
          jax.ShapeDtypeStruct((PH, LB), jnp.float32),
          jax.ShapeDtypeStruct((PH, LB), jnp.float32),
      ],
  )(za_p, zb_p, st, gr, br)


def _pool(h2, batch3d):
  def body(h_ref, bat_ref, gs_ref):
    i = pl.program_id(0)
    valid = (lax.broadcasted_iota(jnp.int32, (BN, 1), 0) + i * BN) < N
    hnm = jnp.where(valid, h_ref[...], 0.0)
    bid = bat_ref[0, 0, :]
    oh = (lax.broadcasted_iota(jnp.int32, (G, BN), 0)
          == bid[None, :]).astype(jnp.float32)
    p = jnp.dot(oh, hnm, preferred_element_type=jnp.float32)

    @pl.when(i == 0)
    def _():
      gs_ref[...] = p

    @pl.when(i > 0)
    def _():
      gs_ref[...] = gs_ref[...] + p

  return pl.pallas_call(
      body,
      grid=(NBP,),
      in_specs=[
          pl.BlockSpec((BN, H), lambda i: (i, 0)),
          pl.BlockSpec((1, 1, BN), lambda i: (i, 0, 0)),
      ],
      out_specs=pl.BlockSpec((G, H), lambda i: (0, 0)),
      out_shape=jax.ShapeDtypeStruct((G, H), jnp.float32),
  )(h2, batch3d)


def _fc(gsum, fcw, fcb):
  def body(g_ref, w_ref, b_ref, o_ref):
    o_ref[...] = (jnp.dot(g_ref[...], w_ref[...],
                          preferred_element_type=jnp.float32) + b_ref[...])

  return pl.pallas_call(
      body,
      out_shape=jax.ShapeDtypeStruct((G, NCLS), jnp.float32),
  )(gsum, fcw, fcb)


def kernel(x, edge_index, batch, params):
  src = edge_index[0]
  dst = edge_index[1]
  npad = E_PAD - E
  pad_src = jnp.arange(npad, dtype=jnp.int32) % LB
  pad_dst = N + jnp.arange(npad, dtype=jnp.int32) % (N_PAD - N)
  src2d = jnp.concatenate([src, pad_src]).reshape(TOTAL_BLOCKS, LB)
  dst2d = jnp.concatenate([dst, pad_dst]).reshape(TOTAL_BLOCKS, LB)

  xa_p = jnp.concatenate(
      [x, jnp.zeros((N, FH - 7), jnp.float32)], axis=1).reshape(PH, LB)
  xb_p = jnp.zeros((PH, LB), jnp.float32)
  zfill = jnp.zeros((RPT, FH), jnp.float32)
  batch3d = jnp.concatenate(
      [batch, jnp.zeros((NBP * BN - N,), batch.dtype)]).reshape(
          NBP, 1, BN).astype(jnp.int32)

  eye8 = jnp.eye(8, dtype=jnp.float32)

  def halves(w):
    waa = w[:FH, :FH]
    wba = jnp.zeros((FH, FH), jnp.float32).at[:H - FH, :].set(w[FH:, :FH])
    wab = jnp.zeros((FH, FH), jnp.float32).at[:, :H - FH].set(w[:FH, FH:])
    wbb = jnp.zeros((FH, FH), jnp.float32).at[:H - FH, :H - FH].set(
        w[FH:, FH:])
    return jnp.stack([jnp.kron(eye8, m) for m in (waa, wba, wab, wbb)])

  def btile(b):
    ba = jnp.tile(b[:FH], 8)
    bb = jnp.tile(jnp.concatenate([b[FH:], jnp.zeros((2 * FH - H,),
                                                     jnp.float32)]), 8)
    return jnp.stack([ba, bb])

  w1p0 = jnp.concatenate(
      [params['W1_0'], jnp.zeros((H - 7, H), jnp.float32)], axis=0)
  kw1s = jnp.stack([halves(w1p0), halves(params['W1_1']),
                    halves(params['W1_2'])])
  kw2s = jnp.stack([halves(params[f'W2_{i}']) for i in range(3)])
  b1s = jnp.stack([btile(params[f'b1_{i}']) for i in range(3)])
  b2s = jnp.stack([btile(params[f'b2_{i}']) for i in range(3)])
  gms = jnp.stack([btile(params[f'bn_gamma_{i}']) for i in range(3)])
  bts = jnp.stack([btile(params[f'bn_beta_{i}']) for i in range(3)])

  agg = _make_sc_agg()

  def layer(h, ws):
    ha_p, hb_p = h
    kw1, b1r, kw2, b2r, gr, br = ws
    a = agg(ha_p.reshape(N, FH), hb_p.reshape(N, FH), src2d, dst2d, zfill)
    za_p, zb_p, st = _dense1(ha_p, hb_p, a.reshape(PAGG, LB),
                             kw1, b1r, kw2, b2r)
    return _bn_relu(za_p, zb_p, st, gr, br), None

  (ha2, hb2), _ = lax.scan(layer, (xa_p, xb_p), (kw1s, b1s, kw2s, b2s,
                                                 gms, bts))

  h2 = jnp.concatenate([ha2.reshape(N, FH),
                        hb2.reshape(N, FH)[:, :H - FH]], axis=1)
  gsum = _pool(h2, batch3d)
  out = _fc(gsum, params['fc_W'], params['fc_b'].reshape(1, NCLS))
  return (out, h2, gsum)


# speedup vs baseline: 1.0089x; 1.0014x over previous
import functools

import jax
import jax.numpy as jnp
from jax import lax
from jax.experimental import pallas as pl
from jax.experimental.pallas import tpu as pltpu
from jax.experimental.pallas import tpu_sc as plsc

N = 100000
E = 3200000
H = 20
G = 512
NCLS = 2

FH = 16
NC = 2
NS = 16
LB = 128
K = 5
TOTAL_BLOCKS = 25120
BPT = TOTAL_BLOCKS // NS
OUTER = BPT // K
E_PAD = TOTAL_BLOCKS * LB
N_PAD = 100352
RPT = N_PAD // NS

PH = N * FH // LB
PB = 448
NB = -(-PH // PB)
PAGG = NC * N_PAD * FH // LB
AOFF = N_PAD * FH // LB
BN = 2048
NBP = -(-N // BN)


def _make_sc_agg():
  mesh = plsc.VectorSubcoreMesh(core_axis_name="c", subcore_axis_name="s")

  @functools.partial(
      pl.kernel,
      mesh=mesh,
      compiler_params=pltpu.CompilerParams(use_tc_tiling_on_sc=False),
      out_type=jax.ShapeDtypeStruct((NC * N_PAD, FH), jnp.float32),
      scratch_types=[
          pltpu.VMEM((3, K, LB), jnp.int32),
          pltpu.VMEM((3, K, LB), jnp.int32),
          pltpu.VMEM((2, K, LB, FH), jnp.float32),
          pltpu.VMEM_SHARED((N_PAD, FH), jnp.float32),
          pltpu.SemaphoreType.DMA,
          pltpu.SemaphoreType.DMA,
      ],
  )
  def agg(ha_hbm, hb_hbm, src_hbm, dst_hbm, zr_hbm, out_hbm,
          src_v, dst_v, rows_v, acc_sh, gsem, isem):
    c = lax.axis_index("c")
    s = lax.axis_index("s")
    row0 = s * RPT

    blk0 = s * BPT

    def run_edges(h_hbm):
      def idx_copy(g, q, sem):
        base = blk0 + g * K
        return (
            pltpu.make_async_copy(src_hbm.at[pl.ds(base, K)],
                                  src_v.at[q], sem),
            pltpu.make_async_copy(dst_hbm.at[pl.ds(base, K)],
                                  dst_v.at[q], sem),
        )

      def gather(q, p):
        return [pltpu.make_async_copy(h_hbm.at[src_v.at[q, j]],
                                      rows_v.at[p, j], gsem)
                for j in range(K)]

      for cp in idx_copy(0, 0, isem):
        cp.start()
        cp.wait()
      cps0 = gather(0, 0)
      for cp in cps0:
        cp.start()
      for cp in idx_copy(1, 1, isem):
        cp.start()
      pltpu.sync_copy(zr_hbm, acc_sh.at[pl.ds(row0, RPT)])
      plsc.subcore_barrier()
      for cp in cps0:
        cp.wait()

      def ebody(g, carry):
        p = lax.rem(g, 2)
        pn = 1 - p
        q0 = lax.rem(g, 3)
        q1 = lax.rem(g + 1, 3)
        q2 = lax.rem(g + 2, 3)

        @pl.when(g + 1 < OUTER)
        def _():
          for cp in idx_copy(g + 1, q1, isem):
            cp.wait()
          for cp in gather(q1, pn):
            cp.start()

        @pl.when(g + 2 < OUTER)
        def _():
          for cp in idx_copy(g + 2, q2, isem):
            cp.start()

        for j in range(K):
          pltpu.sync_copy(rows_v.at[p, j], acc_sh.at[dst_v.at[q0, j]],
                          add=True)

        @pl.when(g + 1 < OUTER)
        def _():
          for cp in gather(q1, pn):
            cp.wait()
        return carry
      lax.fori_loop(0, OUTER, ebody, 0)

    @pl.when(c == 0)
    def _():
      run_edges(ha_hbm)

    @pl.when(c == 1)
    def _():
      run_edges(hb_hbm)

    plsc.subcore_barrier()

    pltpu.sync_copy(acc_sh.at[pl.ds(row0, RPT)],
                    out_hbm.at[pl.ds(c * N_PAD + row0, RPT)])

  return agg


def _dense1(ha_p, hb_p, agg_p, kw1, b1r, kw2, b2r):
  def body(ha_ref, hb_ref, a0_ref, a1_ref, kw1_ref, b1_ref, kw2_ref, b2_ref,
           za_ref, zb_ref, st_ref):
    i = pl.program_id(0)
    pa = ha_ref[...] + a0_ref[...]
    pb = hb_ref[...] + a1_ref[...]

    def mm(xa, xb, kw_ref, b_ref):
      ya = (jnp.dot(xa, kw_ref[0], preferred_element_type=jnp.float32)
            + jnp.dot(xb, kw_ref[1], preferred_element_type=jnp.float32)
            + b_ref[0:1])
      yb = (jnp.dot(xa, kw_ref[2], preferred_element_type=jnp.float32)
            + jnp.dot(xb, kw_ref[3], preferred_element_type=jnp.float32)
            + b_ref[1:2])
      return ya, yb

    z1a, z1b = mm(pa, pb, kw1_ref, b1_ref)
    z1a = jnp.maximum(z1a, 0.0)
    z1b = jnp.maximum(z1b, 0.0)
    za, zb = mm(z1a, z1b, kw2_ref, b2_ref)
    za_ref[...] = za
    zb_ref[...] = zb
    valid = (lax.broadcasted_iota(jnp.int32, (PB, 1), 0) + i * PB) < PH
    zam = jnp.where(valid, za, 0.0)
    zbm = jnp.where(valid, zb, 0.0)
    st = jnp.concatenate(
        [jnp.sum(zam, axis=0, keepdims=True),
         jnp.sum(zam * zam, axis=0, keepdims=True),
         jnp.sum(zbm, axis=0, keepdims=True),
         jnp.sum(zbm * zbm, axis=0, keepdims=True),
         jnp.zeros((4, LB), jnp.float32)], axis=0)

    @pl.when(i == 0)
    def _():
      st_ref[...] = st

    @pl.when(i > 0)
    def _():
      st_ref[...] = st_ref[...] + st

  return pl.pallas_call(
      body,
      grid=(NB,),
      in_specs=[
          pl.BlockSpec((PB, LB), lambda i: (i, 0)),
          pl.BlockSpec((PB, LB), lambda i: (i, 0)),
          pl.BlockSpec((PB, LB), lambda i: (i, 0)),
          pl.BlockSpec((PB, LB), lambda i: (AOFF // PB + i, 0)),
          pl.BlockSpec((4, LB, LB), lambda i: (0, 0, 0)),
          pl.BlockSpec((2, LB), lambda i: (0, 0)),
          pl.BlockSpec((4, LB, LB), lambda i: (0, 0, 0)),
          pl.BlockSpec((2, LB), lambda i: (0, 0)),
      ],
      out_specs=[
          pl.BlockSpec((PB, LB), lambda i: (i, 0)),
          pl.BlockSpec((PB, LB), lambda i: (i, 0)),
          pl.BlockSpec((8, LB), lambda i: (0, 0)),
      ],
      out_shape=[
          jax.ShapeDtypeStruct((PH, LB), jnp.float32),
          jax.ShapeDtypeStruct((PH, LB), jnp.float32),
          jax.ShapeDtypeStruct((8, LB), jnp.float32),
      ],
  )(ha_p, hb_p, agg_p, agg_p, kw1, b1r, kw2, b2r)


def _fold(row):
  t = row[:, 0:16]
  for k in range(1, 8):
    t = t + row[:, 16 * k:16 * k + 16]
  return jnp.concatenate([t] * 8, axis=1)


def _bn_relu(za_p, zb_p, st, gr, br):
  def body(za_ref, zb_ref, st_ref, g_ref, b_ref, ha_ref, hb_ref):
    n_inv = 1.0 / N
    mua = _fold(st_ref[0:1]) * n_inv
    ex2a = _fold(st_ref[1:2]) * n_inv
    mub = _fold(st_ref[2:3]) * n_inv
    ex2b = _fold(st_ref[3:4]) * n_inv
    inva = lax.rsqrt(ex2a - mua * mua + 1e-5)
    invb = lax.rsqrt(ex2b - mub * mub + 1e-5)
    ha_ref[...] = jnp.maximum(
        g_ref[0:1] * (za_ref[...] - mua) * inva + b_ref[0:1], 0.0)
    hb_ref[...] = jnp.maximum(
        g_ref[1:2] * (zb_ref[...] - mub) * invb + b_ref[1:2], 0.0)

  return pl.pallas_call(
      body,
      grid=(NB,),
      in_specs=[
          pl.BlockSpec((PB, LB), lambda i: (i, 0)),
          pl.BlockSpec((PB, LB), lambda i: (i, 0)),
          pl.BlockSpec((8, LB), lambda i: (0, 0)),
          pl.BlockSpec((2, LB), lambda i: (0, 0)),
          pl.BlockSpec((2, LB), lambda i: (0, 0)),
      ],
      out_specs=[
          pl.BlockSpec((PB, LB), lambda i: (i, 0)),
          pl.BlockSpec((PB, LB), lambda i: (i, 0)),
      ],
      out_shape=[
          jax.ShapeDtypeStruct((PH, LB), jnp.float32),
          jax.ShapeDtypeStruct((PH, LB), jnp.float32),
      ],
  )(za_p, zb_p, st, gr, br)


def _pool(h2, batch3d):
  def body(h_ref, bat_ref, gs_ref):
    i = pl.program_id(0)
    valid = (lax.broadcasted_iota(jnp.int32, (BN, 1), 0) + i * BN) < N
    hnm = jnp.where(valid, h_ref[...], 0.0)
    bid = bat_ref[0, 0, :]
    oh = (lax.broadcasted_iota(jnp.int32, (G, BN), 0)
          == bid[None, :]).astype(jnp.float32)
    p = jnp.dot(oh, hnm, preferred_element_type=jnp.float32)

    @pl.when(i == 0)
    def _():
      gs_ref[...] = p

    @pl.when(i > 0)
    def _():
      gs_ref[...] = gs_ref[...] + p

  return pl.pallas_call(
      body,
      grid=(NBP,),
      in_specs=[
          pl.BlockSpec((BN, H), lambda i: (i, 0)),
          pl.BlockSpec((1, 1, BN), lambda i: (i, 0, 0)),
      ],
      out_specs=pl.BlockSpec((G, H), lambda i: (0, 0)),
      out_shape=jax.ShapeDtypeStruct((G, H), jnp.float32),
  )(h2, batch3d)


def _fc(gsum, fcw, fcb):
  def body(g_ref, w_ref, b_ref, o_ref):
    o_ref[...] = (jnp.dot(g_ref[...], w_ref[...],
                          preferred_element_type=jnp.float32) + b_ref[...])

  return pl.pallas_call(
      body,
      out_shape=jax.ShapeDtypeStruct((G, NCLS), jnp.float32),
  )(gsum, fcw, fcb)


def kernel(x, edge_index, batch, params):
  src = edge_index[0]
  dst = edge_index[1]
  npad = E_PAD - E
  pad_src = jnp.arange(npad, dtype=jnp.int32) % LB
  pad_dst = N + jnp.arange(npad, dtype=jnp.int32) % (N_PAD - N)
  src2d = jnp.concatenate([src, pad_src]).reshape(TOTAL_BLOCKS, LB)
  dst2d = jnp.concatenate([dst, pad_dst]).reshape(TOTAL_BLOCKS, LB)

  xa_p = jnp.concatenate(
      [x, jnp.zeros((N, FH - 7), jnp.float32)], axis=1).reshape(PH, LB)
  xb_p = jnp.zeros((PH, LB), jnp.float32)
  zfill = jnp.zeros((RPT, FH), jnp.float32)
  batch3d = jnp.concatenate(
      [batch, jnp.zeros((NBP * BN - N,), batch.dtype)]).reshape(
          NBP, 1, BN).astype(jnp.int32)

  eye8 = jnp.eye(8, dtype=jnp.float32)

  def halves(w):
    waa = w[:FH, :FH]
    wba = jnp.zeros((FH, FH), jnp.float32).at[:H - FH, :].set(w[FH:, :FH])
    wab = jnp.zeros((FH, FH), jnp.float32).at[:, :H - FH].set(w[:FH, FH:])
    wbb = jnp.zeros((FH, FH), jnp.float32).at[:H - FH, :H - FH].set(
        w[FH:, FH:])
    return jnp.stack([jnp.kron(eye8, m) for m in (waa, wba, wab, wbb)])

  def btile(b):
    ba = jnp.tile(b[:FH], 8)
    bb = jnp.tile(jnp.concatenate([b[FH:], jnp.zeros((2 * FH - H,),
                                                     jnp.float32)]), 8)
    return jnp.stack([ba, bb])

  w1p0 = jnp.concatenate(
      [params['W1_0'], jnp.zeros((H - 7, H), jnp.float32)], axis=0)
  kw1s = jnp.stack([halves(w1p0), halves(params['W1_1']),
                    halves(params['W1_2'])])
  kw2s = jnp.stack([halves(params[f'W2_{i}']) for i in range(3)])
  b1s = jnp.stack([btile(params[f'b1_{i}']) for i in range(3)])
  b2s = jnp.stack([btile(params[f'b2_{i}']) for i in range(3)])
  gms = jnp.stack([btile(params[f'bn_gamma_{i}']) for i in range(3)])
  bts = jnp.stack([btile(params[f'bn_beta_{i}']) for i in range(3)])

  agg = _make_sc_agg()

  def layer(h, ws):
    ha_p, hb_p = h
    kw1, b1r, kw2, b2r, gr, br = ws
    a = agg(ha_p.reshape(N, FH), hb_p.reshape(N, FH), src2d, dst2d, zfill)
    za_p, zb_p, st = _dense1(ha_p, hb_p, a.reshape(PAGG, LB),
                             kw1, b1r, kw2, b2r)
    return _bn_relu(za_p, zb_p, st, gr, br), None

  (ha2, hb2), _ = lax.scan(layer, (xa_p, xb_p), (kw1s, b1s, kw2s, b2s,
                                                 gms, bts))

  h2 = jnp.concatenate([ha2.reshape(N, FH),
                        hb2.reshape(N, FH)[:, :H - FH]], axis=1)
  gsum = _pool(h2, batch3d)
  out = _fc(gsum, params['fc_W'], params['fc_b'].reshape(1, NCLS))
  return (out, h2, gsum)


# speedup vs baseline: 1.0353x; 1.0261x over previous
import functools

import jax
import jax.numpy as jnp
from jax import lax
from jax.experimental import pallas as pl
from jax.experimental.pallas import tpu as pltpu
from jax.experimental.pallas import tpu_sc as plsc

N = 100000
E = 3200000
H = 20
G = 512
NCLS = 2

FH = 16
NC = 2
NS = 16
LB = 128
K = 5
TOTAL_BLOCKS = 25120
BPT = TOTAL_BLOCKS // NS
OUTER = BPT // K
E_PAD = TOTAL_BLOCKS * LB
N_PAD = 100352
RPT = N_PAD // NS

PH = N * FH // LB
PB = 896
NB = -(-PH // PB)
PAGG = NC * N_PAD * FH // LB
AOFF = N_PAD * FH // LB
BN = 2048
NBP = -(-N // BN)


def _make_sc_agg():
  mesh = plsc.VectorSubcoreMesh(core_axis_name="c", subcore_axis_name="s")

  @functools.partial(
      pl.kernel,
      mesh=mesh,
      compiler_params=pltpu.CompilerParams(use_tc_tiling_on_sc=False),
      out_type=jax.ShapeDtypeStruct((NC * N_PAD, FH), jnp.float32),
      scratch_types=[
          pltpu.VMEM((3, K, LB), jnp.int32),
          pltpu.VMEM((3, K, LB), jnp.int32),
          pltpu.VMEM((2, K, LB, FH), jnp.float32),
          pltpu.VMEM_SHARED((N_PAD, FH), jnp.float32),
          pltpu.SemaphoreType.DMA,
          pltpu.SemaphoreType.DMA,
      ],
  )
  def agg(ha_hbm, hb_hbm, src_hbm, dst_hbm, zr_hbm, out_hbm,
          src_v, dst_v, rows_v, acc_sh, gsem, isem):
    c = lax.axis_index("c")
    s = lax.axis_index("s")
    row0 = s * RPT

    blk0 = s * BPT

    def run_edges(h_hbm):
      def idx_copy(g, q, sem):
        base = blk0 + g * K
        return (
            pltpu.make_async_copy(src_hbm.at[pl.ds(base, K)],
                                  src_v.at[q], sem),
            pltpu.make_async_copy(dst_hbm.at[pl.ds(base, K)],
                                  dst_v.at[q], sem),
        )

      def gather(q, p):
        return [pltpu.make_async_copy(h_hbm.at[src_v.at[q, j]],
                                      rows_v.at[p, j], gsem)
                for j in range(K)]

      for cp in idx_copy(0, 0, isem):
        cp.start()
        cp.wait()
      cps0 = gather(0, 0)
      for cp in cps0:
        cp.start()
      for cp in idx_copy(1, 1, isem):
        cp.start()
      pltpu.sync_copy(zr_hbm, acc_sh.at[pl.ds(row0, RPT)])
      plsc.subcore_barrier()
      for cp in cps0:
        cp.wait()

      def ebody(g, carry):
        p = lax.rem(g, 2)
        pn = 1 - p
        q0 = lax.rem(g, 3)
        q1 = lax.rem(g + 1, 3)
        q2 = lax.rem(g + 2, 3)

        @pl.when(g + 1 < OUTER)
        def _():
          for cp in idx_copy(g + 1, q1, isem):
            cp.wait()
          for cp in gather(q1, pn):
            cp.start()

        @pl.when(g + 2 < OUTER)
        def _():
          for cp in idx_copy(g + 2, q2, isem):
            cp.start()

        for j in range(K):
          pltpu.sync_copy(rows_v.at[p, j], acc_sh.at[dst_v.at[q0, j]],
                          add=True)

        @pl.when(g + 1 < OUTER)
        def _():
          for cp in gather(q1, pn):
            cp.wait()
        return carry
      lax.fori_loop(0, OUTER, ebody, 0)

    @pl.when(c == 0)
    def _():
      run_edges(ha_hbm)

    @pl.when(c == 1)
    def _():
      run_edges(hb_hbm)

    plsc.subcore_barrier()

    pltpu.sync_copy(acc_sh.at[pl.ds(row0, RPT)],
                    out_hbm.at[pl.ds(c * N_PAD + row0, RPT)])

  return agg


def _dense1(ha_p, hb_p, agg_p, kw1, b1r, kw2, b2r):
  def body(ha_ref, hb_ref, a0_ref, a1_ref, kw1_ref, b1_ref, kw2_ref, b2_ref,
           za_ref, zb_ref, st_ref):
    i = pl.program_id(0)
    pa = ha_ref[...] + a0_ref[...]
    pb = hb_ref[...] + a1_ref[...]

    def mm(xa, xb, kw_ref, b_ref):
      ya = (jnp.dot(xa, kw_ref[0], preferred_element_type=jnp.float32)
            + jnp.dot(xb, kw_ref[1], preferred_element_type=jnp.float32)
            + b_ref[0:1])
      yb = (jnp.dot(xa, kw_ref[2], preferred_element_type=jnp.float32)
            + jnp.dot(xb, kw_ref[3], preferred_element_type=jnp.float32)
            + b_ref[1:2])
      return ya, yb

    z1a, z1b = mm(pa, pb, kw1_ref, b1_ref)
    z1a = jnp.maximum(z1a, 0.0)
    z1b = jnp.maximum(z1b, 0.0)
    za, zb = mm(z1a, z1b, kw2_ref, b2_ref)
    za_ref[...] = za
    zb_ref[...] = zb
    valid = (lax.broadcasted_iota(jnp.int32, (PB, 1), 0) + i * PB) < PH
    zam = jnp.where(valid, za, 0.0)
    zbm = jnp.where(valid, zb, 0.0)
    st = jnp.concatenate(
        [jnp.sum(zam, axis=0, keepdims=True),
         jnp.sum(zam * zam, axis=0, keepdims=True),
         jnp.sum(zbm, axis=0, keepdims=True),
         jnp.sum(zbm * zbm, axis=0, keepdims=True),
         jnp.zeros((4, LB), jnp.float32)], axis=0)

    @pl.when(i == 0)
    def _():
      st_ref[...] = st

    @pl.when(i > 0)
    def _():
      st_ref[...] = st_ref[...] + st

  return pl.pallas_call(
      body,
      grid=(NB,),
      in_specs=[
          pl.BlockSpec((PB, LB), lambda i: (i, 0)),
          pl.BlockSpec((PB, LB), lambda i: (i, 0)),
          pl.BlockSpec((PB, LB), lambda i: (i, 0)),
          pl.BlockSpec((PB, LB), lambda i: (AOFF // PB + i, 0)),
          pl.BlockSpec((4, LB, LB), lambda i: (0, 0, 0)),
          pl.BlockSpec((2, LB), lambda i: (0, 0)),
          pl.BlockSpec((4, LB, LB), lambda i: (0, 0, 0)),
          pl.BlockSpec((2, LB), lambda i: (0, 0)),
      ],
      out_specs=[
          pl.BlockSpec((PB, LB), lambda i: (i, 0)),
          pl.BlockSpec((PB, LB), lambda i: (i, 0)),
          pl.BlockSpec((8, LB), lambda i: (0, 0)),
      ],
      out_shape=[
          jax.ShapeDtypeStruct((PH, LB), jnp.float32),
          jax.ShapeDtypeStruct((PH, LB), jnp.float32),
          jax.ShapeDtypeStruct((8, LB), jnp.float32),
      ],
  )(ha_p, hb_p, agg_p, agg_p, kw1, b1r, kw2, b2r)


def _fold(row):
  t = row[:, 0:16]
  for k in range(1, 8):
    t = t + row[:, 16 * k:16 * k + 16]
  return jnp.concatenate([t] * 8, axis=1)


def _bn_relu(za_p, zb_p, st, gr, br):
  def body(za_ref, zb_ref, st_ref, g_ref, b_ref, ha_ref, hb_ref):
    n_inv = 1.0 / N
    mua = _fold(st_ref[0:1]) * n_inv
    ex2a = _fold(st_ref[1:2]) * n_inv
    mub = _fold(st_ref[2:3]) * n_inv
    ex2b = _fold(st_ref[3:4]) * n_inv
    inva = lax.rsqrt(ex2a - mua * mua + 1e-5)
    invb = lax.rsqrt(ex2b - mub * mub + 1e-5)
    ha_ref[...] = jnp.maximum(
        g_ref[0:1] * (za_ref[...] - mua) * inva + b_ref[0:1], 0.0)
    hb_ref[...] = jnp.maximum(
        g_ref[1:2] * (zb_ref[...] - mub) * invb + b_ref[1:2], 0.0)

  return pl.pallas_call(
      body,
      grid=(NB,),
      in_specs=[
          pl.BlockSpec((PB, LB), lambda i: (i, 0)),
          pl.BlockSpec((PB, LB), lambda i: (i, 0)),
          pl.BlockSpec((8, LB), lambda i: (0, 0)),
          pl.BlockSpec((2, LB), lambda i: (0, 0)),
          pl.BlockSpec((2, LB), lambda i: (0, 0)),
      ],
      out_specs=[
          pl.BlockSpec((PB, LB), lambda i: (i, 0)),
          pl.BlockSpec((PB, LB), lambda i: (i, 0)),
      ],
      out_shape=[
          jax.ShapeDtypeStruct((PH, LB), jnp.float32),
          jax.ShapeDtypeStruct((PH, LB), jnp.float32),
      ],
  )(za_p, zb_p, st, gr, br)


def _pool(h2, batch3d):
  def body(h_ref, bat_ref, gs_ref):
    i = pl.program_id(0)
    valid = (lax.broadcasted_iota(jnp.int32, (BN, 1), 0) + i * BN) < N
    hnm = jnp.where(valid, h_ref[...], 0.0)
    bid = bat_ref[0, 0, :]
    oh = (lax.broadcasted_iota(jnp.int32, (G, BN), 0)
          == bid[None, :]).astype(jnp.bfloat16)
    p = jnp.dot(oh, hnm.astype(jnp.bfloat16),
                preferred_element_type=jnp.float32)

    @pl.when(i == 0)
    def _():
      gs_ref[...] = p

    @pl.when(i > 0)
    def _():
      gs_ref[...] = gs_ref[...] + p

  return pl.pallas_call(
      body,
      grid=(NBP,),
      in_specs=[
          pl.BlockSpec((BN, H), lambda i: (i, 0)),
          pl.BlockSpec((1, 1, BN), lambda i: (i, 0, 0)),
      ],
      out_specs=pl.BlockSpec((G, H), lambda i: (0, 0)),
      out_shape=jax.ShapeDtypeStruct((G, H), jnp.float32),
  )(h2, batch3d)


def _fc(gsum, fcw, fcb):
  def body(g_ref, w_ref, b_ref, o_ref):
    o_ref[...] = (jnp.dot(g_ref[...], w_ref[...],
                          preferred_element_type=jnp.float32) + b_ref[...])

  return pl.pallas_call(
      body,
      out_shape=jax.ShapeDtypeStruct((G, NCLS), jnp.float32),
  )(gsum, fcw, fcb)


def kernel(x, edge_index, batch, params):
  src = edge_index[0]
  dst = edge_index[1]
  npad = E_PAD - E
  pad_src = jnp.arange(npad, dtype=jnp.int32) % LB
  pad_dst = N + jnp.arange(npad, dtype=jnp.int32) % (N_PAD - N)
  src2d = jnp.concatenate([src, pad_src]).reshape(TOTAL_BLOCKS, LB)
  dst2d = jnp.concatenate([dst, pad_dst]).reshape(TOTAL_BLOCKS, LB)

  xa_p = jnp.concatenate(
      [x, jnp.zeros((N, FH - 7), jnp.float32)], axis=1).reshape(PH, LB)
  xb_p = jnp.zeros((PH, LB), jnp.float32)
  zfill = jnp.zeros((RPT, FH), jnp.float32)
  batch3d = jnp.concatenate(
      [batch, jnp.zeros((NBP * BN - N,), batch.dtype)]).reshape(
          NBP, 1, BN).astype(jnp.int32)

  eye8 = jnp.eye(8, dtype=jnp.float32)

  def halves(w):
    waa = w[:FH, :FH]
    wba = jnp.zeros((FH, FH), jnp.float32).at[:H - FH, :].set(w[FH:, :FH])
    wab = jnp.zeros((FH, FH), jnp.float32).at[:, :H - FH].set(w[:FH, FH:])
    wbb = jnp.zeros((FH, FH), jnp.float32).at[:H - FH, :H - FH].set(
        w[FH:, FH:])
    return jnp.stack([jnp.kron(eye8, m) for m in (waa, wba, wab, wbb)])

  def btile(b):
    ba = jnp.tile(b[:FH], 8)
    bb = jnp.tile(jnp.concatenate([b[FH:], jnp.zeros((2 * FH - H,),
                                                     jnp.float32)]), 8)
    return jnp.stack([ba, bb])

  w1p0 = jnp.concatenate(
      [params['W1_0'], jnp.zeros((H - 7, H), jnp.float32)], axis=0)
  kw1s = jnp.stack([halves(w1p0), halves(params['W1_1']),
                    halves(params['W1_2'])])
  kw2s = jnp.stack([halves(params[f'W2_{i}']) for i in range(3)])
  b1s = jnp.stack([btile(params[f'b1_{i}']) for i in range(3)])
  b2s = jnp.stack([btile(params[f'b2_{i}']) for i in range(3)])
  gms = jnp.stack([btile(params[f'bn_gamma_{i}']) for i in range(3)])
  bts = jnp.stack([btile(params[f'bn_beta_{i}']) for i in range(3)])

  agg = _make_sc_agg()

  def layer(h, ws):
    ha_p, hb_p = h
    kw1, b1r, kw2, b2r, gr, br = ws
    a = agg(ha_p.reshape(N, FH), hb_p.reshape(N, FH), src2d, dst2d, zfill)
    za_p, zb_p, st = _dense1(ha_p, hb_p, a.reshape(PAGG, LB),
                             kw1, b1r, kw2, b2r)
    return _bn_relu(za_p, zb_p, st, gr, br), None

  (ha2, hb2), _ = lax.scan(layer, (xa_p, xb_p), (kw1s, b1s, kw2s, b2s,
                                                 gms, bts))

  h2 = jnp.concatenate([ha2.reshape(N, FH),
                        hb2.reshape(N, FH)[:, :H - FH]], axis=1)
  gsum = _pool(h2, batch3d)
  out = _fc(gsum, params['fc_W'], params['fc_b'].reshape(1, NCLS))
  return (out, h2, gsum)


# speedup vs baseline: 1.0547x; 1.0188x over previous
import functools

import jax
import jax.numpy as jnp
from jax import lax
from jax.experimental import pallas as pl
from jax.experimental.pallas import tpu as pltpu
from jax.experimental.pallas import tpu_sc as plsc

N = 100000
E = 3200000
H = 20
G = 512
NCLS = 2

FH = 16
NC = 2
NS = 16
LB = 128
K = 5
TOTAL_BLOCKS = 25120
BPT = TOTAL_BLOCKS // NS
OUTER = BPT // K
E_PAD = TOTAL_BLOCKS * LB
N_PAD = 100352
RPT = N_PAD // NS

PH = N * FH // LB
PB = 1792
NB = -(-PH // PB)
PAGG = NC * N_PAD * FH // LB
AOFF = N_PAD * FH // LB
BN = 4096
NBP = -(-N // BN)


def _make_sc_agg():
  mesh = plsc.VectorSubcoreMesh(core_axis_name="c", subcore_axis_name="s")

  @functools.partial(
      pl.kernel,
      mesh=mesh,
      compiler_params=pltpu.CompilerParams(use_tc_tiling_on_sc=False),
      out_type=jax.ShapeDtypeStruct((NC * N_PAD, FH), jnp.float32),
      scratch_types=[
          pltpu.VMEM((3, K, LB), jnp.int32),
          pltpu.VMEM((3, K, LB), jnp.int32),
          pltpu.VMEM((2, K, LB, FH), jnp.float32),
          pltpu.VMEM_SHARED((N_PAD, FH), jnp.float32),
          pltpu.SemaphoreType.DMA,
          pltpu.SemaphoreType.DMA,
      ],
  )
  def agg(ha_hbm, hb_hbm, src_hbm, dst_hbm, zr_hbm, out_hbm,
          src_v, dst_v, rows_v, acc_sh, gsem, isem):
    c = lax.axis_index("c")
    s = lax.axis_index("s")
    row0 = s * RPT

    blk0 = s * BPT

    def run_edges(h_hbm):
      def idx_copy(g, q, sem):
        base = blk0 + g * K
        return (
            pltpu.make_async_copy(src_hbm.at[pl.ds(base, K)],
                                  src_v.at[q], sem),
            pltpu.make_async_copy(dst_hbm.at[pl.ds(base, K)],
                                  dst_v.at[q], sem),
        )

      def gather(q, p):
        return [pltpu.make_async_copy(h_hbm.at[src_v.at[q, j]],
                                      rows_v.at[p, j], gsem)
                for j in range(K)]

      for cp in idx_copy(0, 0, isem):
        cp.start()
        cp.wait()
      cps0 = gather(0, 0)
      for cp in cps0:
        cp.start()
      for cp in idx_copy(1, 1, isem):
        cp.start()
      pltpu.sync_copy(zr_hbm, acc_sh.at[pl.ds(row0, RPT)])
      plsc.subcore_barrier()
      for cp in cps0:
        cp.wait()

      def ebody(g, carry):
        p = lax.rem(g, 2)
        pn = 1 - p
        q0 = lax.rem(g, 3)
        q1 = lax.rem(g + 1, 3)
        q2 = lax.rem(g + 2, 3)

        @pl.when(g + 1 < OUTER)
        def _():
          for cp in idx_copy(g + 1, q1, isem):
            cp.wait()
          for cp in gather(q1, pn):
            cp.start()

        @pl.when(g + 2 < OUTER)
        def _():
          for cp in idx_copy(g + 2, q2, isem):
            cp.start()

        for j in range(K):
          pltpu.sync_copy(rows_v.at[p, j], acc_sh.at[dst_v.at[q0, j]],
                          add=True)

        @pl.when(g + 1 < OUTER)
        def _():
          for cp in gather(q1, pn):
            cp.wait()
        return carry
      lax.fori_loop(0, OUTER, ebody, 0)

    @pl.when(c == 0)
    def _():
      run_edges(ha_hbm)

    @pl.when(c == 1)
    def _():
      run_edges(hb_hbm)

    plsc.subcore_barrier()

    pltpu.sync_copy(acc_sh.at[pl.ds(row0, RPT)],
                    out_hbm.at[pl.ds(c * N_PAD + row0, RPT)])

  return agg


def _dense1(ha_p, hb_p, agg_p, kw1, b1r, kw2, b2r):
  def body(ha_ref, hb_ref, a0_ref, a1_ref, kw1_ref, b1_ref, kw2_ref, b2_ref,
           za_ref, zb_ref, st_ref):
    i = pl.program_id(0)
    pa = ha_ref[...] + a0_ref[...]
    pb = hb_ref[...] + a1_ref[...]

    def mm(xa, xb, kw_ref, b_ref):
      ya = (jnp.dot(xa, kw_ref[0], preferred_element_type=jnp.float32)
            + jnp.dot(xb, kw_ref[1], preferred_element_type=jnp.float32)
            + b_ref[0:1])
      yb = (jnp.dot(xa, kw_ref[2], preferred_element_type=jnp.float32)
            + jnp.dot(xb, kw_ref[3], preferred_element_type=jnp.float32)
            + b_ref[1:2])
      return ya, yb

    z1a, z1b = mm(pa, pb, kw1_ref, b1_ref)
    z1a = jnp.maximum(z1a, 0.0)
    z1b = jnp.maximum(z1b, 0.0)
    za, zb = mm(z1a, z1b, kw2_ref, b2_ref)
    za_ref[...] = za
    zb_ref[...] = zb
    valid = (lax.broadcasted_iota(jnp.int32, (PB, 1), 0) + i * PB) < PH
    zam = jnp.where(valid, za, 0.0)
    zbm = jnp.where(valid, zb, 0.0)
    st = jnp.concatenate(
        [jnp.sum(zam, axis=0, keepdims=True),
         jnp.sum(zam * zam, axis=0, keepdims=True),
         jnp.sum(zbm, axis=0, keepdims=True),
         jnp.sum(zbm * zbm, axis=0, keepdims=True),
         jnp.zeros((4, LB), jnp.float32)], axis=0)

    @pl.when(i == 0)
    def _():
      st_ref[...] = st

    @pl.when(i > 0)
    def _():
      st_ref[...] = st_ref[...] + st

  return pl.pallas_call(
      body,
      grid=(NB,),
      in_specs=[
          pl.BlockSpec((PB, LB), lambda i: (i, 0)),
          pl.BlockSpec((PB, LB), lambda i: (i, 0)),
          pl.BlockSpec((PB, LB), lambda i: (i, 0)),
          pl.BlockSpec((PB, LB), lambda i: (AOFF // PB + i, 0)),
          pl.BlockSpec((4, LB, LB), lambda i: (0, 0, 0)),
          pl.BlockSpec((2, LB), lambda i: (0, 0)),
          pl.BlockSpec((4, LB, LB), lambda i: (0, 0, 0)),
          pl.BlockSpec((2, LB), lambda i: (0, 0)),
      ],
      out_specs=[
          pl.BlockSpec((PB, LB), lambda i: (i, 0)),
          pl.BlockSpec((PB, LB), lambda i: (i, 0)),
          pl.BlockSpec((8, LB), lambda i: (0, 0)),
      ],
      out_shape=[
          jax.ShapeDtypeStruct((PH, LB), jnp.float32),
          jax.ShapeDtypeStruct((PH, LB), jnp.float32),
          jax.ShapeDtypeStruct((8, LB), jnp.float32),
      ],
  )(ha_p, hb_p, agg_p, agg_p, kw1, b1r, kw2, b2r)


def _fold(row):
  t = row[:, 0:16]
  for k in range(1, 8):
    t = t + row[:, 16 * k:16 * k + 16]
  return jnp.concatenate([t] * 8, axis=1)


def _bn_relu(za_p, zb_p, st, gr, br):
  def body(za_ref, zb_ref, st_ref, g_ref, b_ref, ha_ref, hb_ref):
    n_inv = 1.0 / N
    mua = _fold(st_ref[0:1]) * n_inv
    ex2a = _fold(st_ref[1:2]) * n_inv
    mub = _fold(st_ref[2:3]) * n_inv
    ex2b = _fold(st_ref[3:4]) * n_inv
    inva = lax.rsqrt(ex2a - mua * mua + 1e-5)
    invb = lax.rsqrt(ex2b - mub * mub + 1e-5)
    ha_ref[...] = jnp.maximum(
        g_ref[0:1] * (za_ref[...] - mua) * inva + b_ref[0:1], 0.0)
    hb_ref[...] = jnp.maximum(
        g_ref[1:2] * (zb_ref[...] - mub) * invb + b_ref[1:2], 0.0)

  return pl.pallas_call(
      body,
      grid=(NB,),
      in_specs=[
          pl.BlockSpec((PB, LB), lambda i: (i, 0)),
          pl.BlockSpec((PB, LB), lambda i: (i, 0)),
          pl.BlockSpec((8, LB), lambda i: (0, 0)),
          pl.BlockSpec((2, LB), lambda i: (0, 0)),
          pl.BlockSpec((2, LB), lambda i: (0, 0)),
      ],
      out_specs=[
          pl.BlockSpec((PB, LB), lambda i: (i, 0)),
          pl.BlockSpec((PB, LB), lambda i: (i, 0)),
      ],
      out_shape=[
          jax.ShapeDtypeStruct((PH, LB), jnp.float32),
          jax.ShapeDtypeStruct((PH, LB), jnp.float32),
      ],
  )(za_p, zb_p, st, gr, br)


def _pool(h2, batch3d):
  def body(h_ref, bat_ref, gs_ref):
    i = pl.program_id(0)
    valid = (lax.broadcasted_iota(jnp.int32, (BN, 1), 0) + i * BN) < N
    hnm = jnp.where(valid, h_ref[...], 0.0)
    bid = bat_ref[0, 0, :]
    oh = (lax.broadcasted_iota(jnp.int32, (G, BN), 0)
          == bid[None, :]).astype(jnp.bfloat16)
    p = jnp.dot(oh, hnm.astype(jnp.bfloat16),
                preferred_element_type=jnp.float32)

    @pl.when(i == 0)
    def _():
      gs_ref[...] = p

    @pl.when(i > 0)
    def _():
      gs_ref[...] = gs_ref[...] + p

  return pl.pallas_call(
      body,
      grid=(NBP,),
      in_specs=[
          pl.BlockSpec((BN, H), lambda i: (i, 0)),
          pl.BlockSpec((1, 1, BN), lambda i: (i, 0, 0)),
      ],
      out_specs=pl.BlockSpec((G, H), lambda i: (0, 0)),
      out_shape=jax.ShapeDtypeStruct((G, H), jnp.float32),
  )(h2, batch3d)


def _fc(gsum, fcw, fcb):
  def body(g_ref, w_ref, b_ref, o_ref):
    o_ref[...] = (jnp.dot(g_ref[...], w_ref[...],
                          preferred_element_type=jnp.float32) + b_ref[...])

  return pl.pallas_call(
      body,
      out_shape=jax.ShapeDtypeStruct((G, NCLS), jnp.float32),
  )(gsum, fcw, fcb)


def kernel(x, edge_index, batch, params):
  src = edge_index[0]
  dst = edge_index[1]
  npad = E_PAD - E
  pad_src = jnp.arange(npad, dtype=jnp.int32) % LB
  pad_dst = N + jnp.arange(npad, dtype=jnp.int32) % (N_PAD - N)
  src2d = jnp.concatenate([src, pad_src]).reshape(TOTAL_BLOCKS, LB)
  dst2d = jnp.concatenate([dst, pad_dst]).reshape(TOTAL_BLOCKS, LB)

  xa_p = jnp.concatenate(
      [x, jnp.zeros((N, FH - 7), jnp.float32)], axis=1).reshape(PH, LB)
  xb_p = jnp.zeros((PH, LB), jnp.float32)
  zfill = jnp.zeros((RPT, FH), jnp.float32)
  batch3d = jnp.concatenate(
      [batch, jnp.zeros((NBP * BN - N,), batch.dtype)]).reshape(
          NBP, 1, BN).astype(jnp.int32)

  eye8 = jnp.eye(8, dtype=jnp.float32)

  def halves(w):
    waa = w[:FH, :FH]
    wba = jnp.zeros((FH, FH), jnp.float32).at[:H - FH, :].set(w[FH:, :FH])
    wab = jnp.zeros((FH, FH), jnp.float32).at[:, :H - FH].set(w[:FH, FH:])
    wbb = jnp.zeros((FH, FH), jnp.float32).at[:H - FH, :H - FH].set(
        w[FH:, FH:])
    return jnp.stack([jnp.kron(eye8, m) for m in (waa, wba, wab, wbb)])

  def btile(b):
    ba = jnp.tile(b[:FH], 8)
    bb = jnp.tile(jnp.concatenate([b[FH:], jnp.zeros((2 * FH - H,),
                                                     jnp.float32)]), 8)
    return jnp.stack([ba, bb])

  w1p0 = jnp.concatenate(
      [params['W1_0'], jnp.zeros((H - 7, H), jnp.float32)], axis=0)
  kw1s = jnp.stack([halves(w1p0), halves(params['W1_1']),
                    halves(params['W1_2'])])
  kw2s = jnp.stack([halves(params[f'W2_{i}']) for i in range(3)])
  b1s = jnp.stack([btile(params[f'b1_{i}']) for i in range(3)])
  b2s = jnp.stack([btile(params[f'b2_{i}']) for i in range(3)])
  gms = jnp.stack([btile(params[f'bn_gamma_{i}']) for i in range(3)])
  bts = jnp.stack([btile(params[f'bn_beta_{i}']) for i in range(3)])

  agg = _make_sc_agg()

  def layer(h, ws):
    ha_p, hb_p = h
    kw1, b1r, kw2, b2r, gr, br = ws
    a = agg(ha_p.reshape(N, FH), hb_p.reshape(N, FH), src2d, dst2d, zfill)
    za_p, zb_p, st = _dense1(ha_p, hb_p, a.reshape(PAGG, LB),
                             kw1, b1r, kw2, b2r)
    return _bn_relu(za_p, zb_p, st, gr, br), None

  (ha2, hb2), _ = lax.scan(layer, (xa_p, xb_p), (kw1s, b1s, kw2s, b2s,
                                                 gms, bts))

  h2 = jnp.concatenate([ha2.reshape(N, FH),
                        hb2.reshape(N, FH)[:, :H - FH]], axis=1)
  gsum = _pool(h2, batch3d)
  out = _fc(gsum, params['fc_W'], params['fc_b'].reshape(1, NCLS))
  return (out, h2, gsum)
